# Initial kernel scaffold; baseline (speedup 1.0000x reference)
import jax
import jax.numpy as jnp
from jax.experimental import pallas as pl


def kernel(x, emb):
    raise NotImplementedError("write your pallas kernel here")


# speedup vs baseline: 1.6349x; 1.6349x over previous
import jax
import jax.numpy as jnp
from jax.experimental import pallas as pl


BATCH = 4
SEQ_LEN = 2048
D_MODEL = 1024
SEQ_BLK = 512


def _add_kernel(x_ref, emb_ref, out_ref):
    out_ref[...] = x_ref[...] + emb_ref[...]


def kernel(x, emb):
    grid = (BATCH, SEQ_LEN // SEQ_BLK)
    return pl.pallas_call(
        _add_kernel,
        grid=grid,
        in_specs=[
            pl.BlockSpec((1, SEQ_BLK, D_MODEL), lambda b, s: (b, s, 0)),
            pl.BlockSpec((SEQ_BLK, D_MODEL), lambda b, s: (s, 0)),
        ],
        out_specs=pl.BlockSpec((1, SEQ_BLK, D_MODEL), lambda b, s: (b, s, 0)),
        out_shape=jax.ShapeDtypeStruct((BATCH, SEQ_LEN, D_MODEL), x.dtype),
    )(x, emb)


# speedup vs baseline: 1.9356x; 1.1840x over previous
import jax
import jax.numpy as jnp
from jax.experimental import pallas as pl


BATCH = 4
SEQ_LEN = 2048
D_MODEL = 1024
SEQ_BLK = 512


def _add_kernel(x_ref, emb_ref, out_ref):
    out_ref[...] = x_ref[...] + emb_ref[...]


def kernel(x, emb):
    grid = (SEQ_LEN // SEQ_BLK, BATCH)
    return pl.pallas_call(
        _add_kernel,
        grid=grid,
        in_specs=[
            pl.BlockSpec((1, SEQ_BLK, D_MODEL), lambda s, b: (b, s, 0)),
            pl.BlockSpec((SEQ_BLK, D_MODEL), lambda s, b: (s, 0)),
        ],
        out_specs=pl.BlockSpec((1, SEQ_BLK, D_MODEL), lambda s, b: (b, s, 0)),
        out_shape=jax.ShapeDtypeStruct((BATCH, SEQ_LEN, D_MODEL), x.dtype),
    )(x, emb)


# speedup vs baseline: 2.1605x; 1.1162x over previous
import jax
import jax.numpy as jnp
from jax.experimental import pallas as pl


BATCH = 4
SEQ_LEN = 2048
D_MODEL = 1024
SEQ_BLK = 512


def _add_kernel(x_ref, emb_ref, out_ref):
    out_ref[...] = x_ref[...] + emb_ref[...][None, :, :]


def kernel(x, emb):
    grid = (SEQ_LEN // SEQ_BLK,)
    return pl.pallas_call(
        _add_kernel,
        grid=grid,
        in_specs=[
            pl.BlockSpec((BATCH, SEQ_BLK, D_MODEL), lambda s: (0, s, 0)),
            pl.BlockSpec((SEQ_BLK, D_MODEL), lambda s: (s, 0)),
        ],
        out_specs=pl.BlockSpec((BATCH, SEQ_BLK, D_MODEL), lambda s: (0, s, 0)),
        out_shape=jax.ShapeDtypeStruct((BATCH, SEQ_LEN, D_MODEL), x.dtype),
    )(x, emb)
